# baseline (speedup 1.0000x reference)
import functools

import jax
import jax.numpy as jnp
from jax import lax
from jax.experimental import pallas as pl
from jax.experimental.pallas import tpu as pltpu
from jax.experimental.pallas import tpu_sc as plsc

_B = 16384
_D = 128
_HID = 64
_OUT = 32
_NC = 2
_NS = 16
_NW = _NC * _NS
_BPW = _B // _NW
_CH = 128
_NCH = _BPW // _CH


def _gather_body(uemb, iemb, uidx, iidx, urows, irows,
                 idx_u, idx_i, buf_u, buf_v,
                 su0, su1, si0, si1):
    cid = lax.axis_index("c")
    sid = lax.axis_index("s")
    wid = sid * _NC + cid
    base = wid * _BPW
    pltpu.sync_copy(uidx.at[pl.ds(wid * _NCH, _NCH)], idx_u)
    pltpu.sync_copy(iidx.at[pl.ds(wid * _NCH, _NCH)], idx_i)
    sems_u = (su0, su1)
    sems_i = (si0, si1)
    gu = [None] * _NCH
    gi = [None] * _NCH
    for c in range(2):
        gu[c] = pltpu.async_copy(uemb.at[idx_u.at[c]], buf_u.at[c % 2], sems_u[c % 2])
        gi[c] = pltpu.async_copy(iemb.at[idx_i.at[c]], buf_v.at[c % 2], sems_i[c % 2])
    for c in range(_NCH):
        gu[c].wait()
        pltpu.sync_copy(buf_u.at[c % 2], urows.at[pl.ds(base + c * _CH, _CH)])
        if c + 2 < _NCH:
            gu[c + 2] = pltpu.async_copy(uemb.at[idx_u.at[c + 2]], buf_u.at[c % 2],
                                         sems_u[c % 2])
        gi[c].wait()
        pltpu.sync_copy(buf_v.at[c % 2], irows.at[pl.ds(base + c * _CH, _CH)])
        if c + 2 < _NCH:
            gi[c + 2] = pltpu.async_copy(iemb.at[idx_i.at[c + 2]], buf_v.at[c % 2],
                                         sems_i[c % 2])


def _sc_gather(uemb, iemb, uidx, iidx):
    mesh = plsc.VectorSubcoreMesh(core_axis_name="c", subcore_axis_name="s",
                                  num_cores=_NC, num_subcores=_NS)
    fn = pl.kernel(
        _gather_body,
        out_type=[jax.ShapeDtypeStruct((_B, _D), jnp.float32),
                  jax.ShapeDtypeStruct((_B, _D), jnp.float32)],
        mesh=mesh,
        scratch_types=[
            pltpu.VMEM((_NCH, _CH), jnp.int32),
            pltpu.VMEM((_NCH, _CH), jnp.int32),
            pltpu.VMEM((2, _CH, _D), jnp.float32),
            pltpu.VMEM((2, _CH, _D), jnp.float32),
            pltpu.SemaphoreType.DMA,
            pltpu.SemaphoreType.DMA,
            pltpu.SemaphoreType.DMA,
            pltpu.SemaphoreType.DMA,
        ],
    )
    return fn(uemb, iemb, uidx, iidx)


_BB = 2048


def _mlp_body(ur, ir, w1, b1, w2, b2, out):
    x = jnp.concatenate([ur[...], ir[...]], axis=1)
    h = jnp.maximum(jnp.dot(x, w1[...], preferred_element_type=jnp.float32)
                    + b1[...], 0.0)
    e = jnp.dot(h, w2[...], preferred_element_type=jnp.float32) + b2[...]
    out[...] = jnp.sum(e[:, :_OUT] * e[:, _OUT:], axis=1)


def _tc_mlp(urows, irows, w1, b1, w2, b2):
    grid = (_B // _BB,)
    full = lambda shape: pl.BlockSpec(shape, lambda b: (0,) * len(shape))
    return pl.pallas_call(
        _mlp_body,
        grid=grid,
        in_specs=[
            pl.BlockSpec((_BB, _D), lambda b: (b, 0)),
            pl.BlockSpec((_BB, _D), lambda b: (b, 0)),
            full((2 * _D, 2 * _HID)), full((1, 2 * _HID)),
            full((2 * _HID, 2 * _OUT)), full((1, 2 * _OUT)),
        ],
        out_specs=pl.BlockSpec((_BB,), lambda b: (b,)),
        out_shape=jax.ShapeDtypeStruct((_B,), jnp.float32),
    )(urows, irows, w1, b1, w2, b2)


def kernel(u, i, user_emb, user_W1, user_b1, user_W2, user_b2,
           item_emb, item_W1, item_b1, item_W2, item_b2):
    uidx = u.astype(jnp.int32).reshape(_NW * _NCH, _CH)
    iidx = i.astype(jnp.int32).reshape(_NW * _NCH, _CH)
    urows, irows = _sc_gather(user_emb, item_emb, uidx, iidx)
    z1 = jnp.zeros((_D, _HID), jnp.float32)
    w1 = jnp.block([[user_W1.T, z1], [z1, item_W1.T]])
    z2 = jnp.zeros((_HID, _OUT), jnp.float32)
    w2 = jnp.block([[user_W2.T, z2], [z2, item_W2.T]])
    b1 = jnp.concatenate([user_b1, item_b1]).reshape(1, 2 * _HID)
    b2 = jnp.concatenate([user_b2, item_b2]).reshape(1, 2 * _OUT)
    return _tc_mlp(urows, irows, w1, b1, w2, b2)


# speedup vs baseline: 1.3305x; 1.3305x over previous
import functools

import jax
import jax.numpy as jnp
from jax import lax
from jax.experimental import pallas as pl
from jax.experimental.pallas import tpu as pltpu
from jax.experimental.pallas import tpu_sc as plsc

_B = 16384
_D = 128
_HID = 64
_OUT = 32
_NC = 2
_NS = 16
_NW = _NC * _NS
_BPW = _B // _NW
_CH = 128
_NCH = _BPW // _CH


def _gather_body(uemb, iemb, uidx, iidx, urows, irows,
                 idx_u, idx_i, buf_u, buf_v,
                 su0, su1, si0, si1):
    cid = lax.axis_index("c")
    sid = lax.axis_index("s")
    wid = sid * _NC + cid
    base = wid * _BPW
    pltpu.sync_copy(uidx.at[pl.ds(wid * _NCH, _NCH)], idx_u)
    pltpu.sync_copy(iidx.at[pl.ds(wid * _NCH, _NCH)], idx_i)
    sems_u = (su0, su1)
    sems_i = (si0, si1)
    gu = [None] * _NCH
    gi = [None] * _NCH
    for c in range(2):
        gu[c] = pltpu.async_copy(uemb.at[idx_u.at[c]], buf_u.at[c % 2], sems_u[c % 2])
        gi[c] = pltpu.async_copy(iemb.at[idx_i.at[c]], buf_v.at[c % 2], sems_i[c % 2])
    for c in range(_NCH):
        gu[c].wait()
        pltpu.sync_copy(buf_u.at[c % 2], urows.at[pl.ds(base + c * _CH, _CH)])
        if c + 2 < _NCH:
            gu[c + 2] = pltpu.async_copy(uemb.at[idx_u.at[c + 2]], buf_u.at[c % 2],
                                         sems_u[c % 2])
        gi[c].wait()
        pltpu.sync_copy(buf_v.at[c % 2], irows.at[pl.ds(base + c * _CH, _CH)])
        if c + 2 < _NCH:
            gi[c + 2] = pltpu.async_copy(iemb.at[idx_i.at[c + 2]], buf_v.at[c % 2],
                                         sems_i[c % 2])


def _sc_gather(uemb, iemb, uidx, iidx):
    mesh = plsc.VectorSubcoreMesh(core_axis_name="c", subcore_axis_name="s",
                                  num_cores=_NC, num_subcores=_NS)
    fn = pl.kernel(
        _gather_body,
        out_type=[jax.ShapeDtypeStruct((_B, _D), jnp.float32),
                  jax.ShapeDtypeStruct((_B, _D), jnp.float32)],
        mesh=mesh,
        scratch_types=[
            pltpu.VMEM((_NCH, _CH), jnp.int32),
            pltpu.VMEM((_NCH, _CH), jnp.int32),
            pltpu.VMEM((2, _CH, _D), jnp.float32),
            pltpu.VMEM((2, _CH, _D), jnp.float32),
            pltpu.SemaphoreType.DMA,
            pltpu.SemaphoreType.DMA,
            pltpu.SemaphoreType.DMA,
            pltpu.SemaphoreType.DMA,
        ],
    )
    return fn(uemb, iemb, uidx, iidx)


_BB = 2048


def _mlp_body(ur, ir, w1, b1, w2, b2, hot, grp, ones_r, out):
    x = jnp.concatenate([ur[...], ir[...]], axis=1)
    h = jnp.maximum(jnp.dot(x, w1[...], preferred_element_type=jnp.float32)
                    + b1[...], 0.0)
    e = jnp.dot(h, w2[...], preferred_element_type=jnp.float32) + b2[...]
    p = e[:, :_OUT] * e[:, _OUT:]
    r = jnp.dot(p, ones_r[...], preferred_element_type=jnp.float32)
    rm = r * hot[...]
    out[...] = jax.lax.dot_general(
        grp[...], rm, (((0,), (0,)), ((), ())),
        preferred_element_type=jnp.float32)


def _tc_mlp(urows, irows, w1, b1, w2, b2, hot, grp, ones_r):
    grid = (_B // _BB,)
    full = lambda shape: pl.BlockSpec(shape, lambda b: (0,) * len(shape))
    return pl.pallas_call(
        _mlp_body,
        grid=grid,
        in_specs=[
            pl.BlockSpec((_BB, _D), lambda b: (b, 0)),
            pl.BlockSpec((_BB, _D), lambda b: (b, 0)),
            full((2 * _D, 2 * _HID)), full((1, 2 * _HID)),
            full((2 * _HID, 2 * _OUT)), full((1, 2 * _OUT)),
            full((_BB, 128)), full((_BB, _BB // 128)), full((_OUT, 128)),
        ],
        out_specs=pl.BlockSpec((_BB // 128, 128), lambda b: (b, 0)),
        out_shape=jax.ShapeDtypeStruct((_B // 128, 128), jnp.float32),
    )(urows, irows, w1, b1, w2, b2, hot, grp, ones_r)


def kernel(u, i, user_emb, user_W1, user_b1, user_W2, user_b2,
           item_emb, item_W1, item_b1, item_W2, item_b2):
    uidx = u.astype(jnp.int32).reshape(_NW * _NCH, _CH)
    iidx = i.astype(jnp.int32).reshape(_NW * _NCH, _CH)
    urows, irows = _sc_gather(user_emb, item_emb, uidx, iidx)
    z1 = jnp.zeros((_D, _HID), jnp.float32)
    w1 = jnp.block([[user_W1.T, z1], [z1, item_W1.T]])
    z2 = jnp.zeros((_HID, _OUT), jnp.float32)
    w2 = jnp.block([[user_W2.T, z2], [z2, item_W2.T]])
    b1 = jnp.concatenate([user_b1, item_b1]).reshape(1, 2 * _HID)
    b2 = jnp.concatenate([user_b2, item_b2]).reshape(1, 2 * _OUT)
    hot = jnp.tile(jnp.eye(128, dtype=jnp.float32), (_BB // 128, 1))
    grp = jnp.repeat(jnp.eye(_BB // 128, dtype=jnp.float32), 128, axis=0)
    ones_r = jnp.ones((_OUT, 128), jnp.float32)
    out2d = _tc_mlp(urows, irows, w1, b1, w2, b2, hot, grp, ones_r)
    return out2d.reshape(_B)


# speedup vs baseline: 1.4375x; 1.0804x over previous
import functools

import jax
import jax.numpy as jnp
import numpy as np
from jax import lax
from jax.experimental import pallas as pl
from jax.experimental.pallas import tpu as pltpu
from jax.experimental.pallas import tpu_sc as plsc

_B = 16384
_D = 128
_HID = 64
_OUT = 32
_NC = 2
_NS = 16
_NW = _NC * _NS
_BPW = _B // _NW
_CH = 128
_NCH = _BPW // _CH


def _gather_body(uemb, iemb, uidx, iidx, urows, irows,
                 idx_u, idx_i, buf_u, buf_v,
                 su0, su1, si0, si1):
    cid = lax.axis_index("c")
    sid = lax.axis_index("s")
    wid = sid * _NC + cid
    base = wid * _BPW
    pltpu.sync_copy(uidx.at[pl.ds(wid * _NCH, _NCH)], idx_u)
    pltpu.sync_copy(iidx.at[pl.ds(wid * _NCH, _NCH)], idx_i)
    sems_u = (su0, su1)
    sems_i = (si0, si1)
    gu = [None] * _NCH
    gi = [None] * _NCH
    for c in range(2):
        gu[c] = pltpu.async_copy(uemb.at[idx_u.at[c]], buf_u.at[c % 2], sems_u[c % 2])
        gi[c] = pltpu.async_copy(iemb.at[idx_i.at[c]], buf_v.at[c % 2], sems_i[c % 2])
    for c in range(_NCH):
        gu[c].wait()
        pltpu.sync_copy(buf_u.at[c % 2], urows.at[pl.ds(base + c * _CH, _CH)])
        if c + 2 < _NCH:
            gu[c + 2] = pltpu.async_copy(uemb.at[idx_u.at[c + 2]], buf_u.at[c % 2],
                                         sems_u[c % 2])
        gi[c].wait()
        pltpu.sync_copy(buf_v.at[c % 2], irows.at[pl.ds(base + c * _CH, _CH)])
        if c + 2 < _NCH:
            gi[c + 2] = pltpu.async_copy(iemb.at[idx_i.at[c + 2]], buf_v.at[c % 2],
                                         sems_i[c % 2])


def _sc_gather(uemb, iemb, uidx, iidx):
    mesh = plsc.VectorSubcoreMesh(core_axis_name="c", subcore_axis_name="s",
                                  num_cores=_NC, num_subcores=_NS)
    fn = pl.kernel(
        _gather_body,
        out_type=[jax.ShapeDtypeStruct((_B, _D), jnp.float32),
                  jax.ShapeDtypeStruct((_B, _D), jnp.float32)],
        mesh=mesh,
        scratch_types=[
            pltpu.VMEM((_NCH, _CH), jnp.int32),
            pltpu.VMEM((_NCH, _CH), jnp.int32),
            pltpu.VMEM((2, _CH, _D), jnp.float32),
            pltpu.VMEM((2, _CH, _D), jnp.float32),
            pltpu.SemaphoreType.DMA,
            pltpu.SemaphoreType.DMA,
            pltpu.SemaphoreType.DMA,
            pltpu.SemaphoreType.DMA,
        ],
    )
    return fn(uemb, iemb, uidx, iidx)


_BB = 4096


def _mlp_body(ur, ir, w1, b1, w2, b2, hot, grp, ones_r, out):
    x = jnp.concatenate([ur[...], ir[...]], axis=1)
    h = jnp.maximum(jnp.dot(x, w1[...], preferred_element_type=jnp.float32)
                    + b1[...], 0.0)
    e = jnp.dot(h, w2[...], preferred_element_type=jnp.float32) + b2[...]
    p = e[:, :_OUT] * e[:, _OUT:]
    r = jnp.dot(p, ones_r[...], preferred_element_type=jnp.float32)
    rm = r * hot[...]
    out[...] = jax.lax.dot_general(
        grp[...], rm, (((0,), (0,)), ((), ())),
        preferred_element_type=jnp.float32)


def _tc_mlp(urows, irows, w1, b1, w2, b2, hot, grp, ones_r):
    grid = (_B // _BB,)
    full = lambda shape: pl.BlockSpec(shape, lambda b: (0,) * len(shape))
    return pl.pallas_call(
        _mlp_body,
        grid=grid,
        in_specs=[
            pl.BlockSpec((_BB, _D), lambda b: (b, 0)),
            pl.BlockSpec((_BB, _D), lambda b: (b, 0)),
            full((2 * _D, 2 * _HID)), full((1, 2 * _HID)),
            full((2 * _HID, 2 * _OUT)), full((1, 2 * _OUT)),
            full((_BB, 128)), full((_BB, _BB // 128)), full((_OUT, 128)),
        ],
        out_specs=pl.BlockSpec((_BB // 128, 128), lambda b: (b, 0)),
        out_shape=jax.ShapeDtypeStruct((_B // 128, 128), jnp.float32),
    )(urows, irows, w1, b1, w2, b2, hot, grp, ones_r)


def kernel(u, i, user_emb, user_W1, user_b1, user_W2, user_b2,
           item_emb, item_W1, item_b1, item_W2, item_b2):
    uidx = u.astype(jnp.int32).reshape(_NW * _NCH, _CH)
    iidx = i.astype(jnp.int32).reshape(_NW * _NCH, _CH)
    urows, irows = _sc_gather(user_emb, item_emb, uidx, iidx)
    z1 = jnp.zeros((_D, _HID), jnp.float32)
    w1 = jnp.block([[user_W1.T, z1], [z1, item_W1.T]])
    z2 = jnp.zeros((_HID, _OUT), jnp.float32)
    w2 = jnp.block([[user_W2.T, z2], [z2, item_W2.T]])
    b1 = jnp.concatenate([user_b1, item_b1]).reshape(1, 2 * _HID)
    b2 = jnp.concatenate([user_b2, item_b2]).reshape(1, 2 * _OUT)
    hot = np.tile(np.eye(128, dtype=np.float32), (_BB // 128, 1))
    grp = np.repeat(np.eye(_BB // 128, dtype=np.float32), 128, axis=0)
    ones_r = np.ones((_OUT, 128), np.float32)
    out2d = _tc_mlp(urows, irows, w1, b1, w2, b2, hot, grp, ones_r)
    return out2d.reshape(_B)
